# baseline (speedup 1.0000x reference)
import functools

import jax
import jax.numpy as jnp
from jax import lax
from jax.experimental import pallas as pl
from jax.experimental.pallas import tpu as pltpu
from jax.experimental.pallas import tpu_sc as plsc

N = 10000
E = 320000
H = 128
C = 10

NC = 2
NS = 16
NW = NC * NS
HH = H // NC
EPW = E // NS
CH = 80
NCHUNK = 252
NBUF = 6
PAD = NCHUNK * CH - EPW
NP = 10240
RPT = NP // NS


def _sc_aggr_body(x_hbm, src_hbm, dst_hbm, zeros_hbm, out_hbm,
                  src_v, dst_v, rows_v, aggr_sh, gsem, ssem):
    c = lax.axis_index("c")
    s = lax.axis_index("s")
    wid = c * NS + s

    pltpu.async_copy(src_hbm.at[wid], src_v, gsem)
    pltpu.async_copy(dst_hbm.at[s], dst_v, gsem)
    pltpu.sync_copy(zeros_hbm, aggr_sh.at[pl.ds(s * RPT, RPT)])
    pltpu.make_async_copy(src_hbm.at[wid], src_v, gsem).wait()
    pltpu.make_async_copy(dst_hbm.at[s], dst_v, gsem).wait()

    plsc.subcore_barrier()

    def start_gather(j, b):
        pltpu.async_copy(x_hbm.at[src_v.at[j]], rows_v.at[b], gsem)

    def wait_gather(b):
        pltpu.make_async_copy(x_hbm.at[pl.ds(0, CH)], rows_v.at[b], gsem).wait()

    def start_scatter(j, b):
        pltpu.async_copy(rows_v.at[b], aggr_sh.at[dst_v.at[j]], ssem, add=True)

    def wait_scatter(b):
        pltpu.make_async_copy(x_hbm.at[pl.ds(0, CH)], rows_v.at[b], ssem).wait()

    for b0 in range(NBUF - 1):
        start_gather(b0, b0)

    def outer(i, carry):
        for b in range(NBUF):
            j = NBUF * i + b
            wait_gather(b)
            start_scatter(j, b)
            jj = j + NBUF - 1

            @pl.when(jnp.logical_and(j >= 1, jj < NCHUNK))
            def _():
                wait_scatter((b + NBUF - 1) % NBUF)

            @pl.when(jj < NCHUNK)
            def _():
                start_gather(jj, (b + NBUF - 1) % NBUF)
        return carry

    lax.fori_loop(0, NCHUNK // NBUF, outer, 0)
    for _ in range(NBUF):
        wait_scatter(0)

    plsc.subcore_barrier()

    pltpu.sync_copy(aggr_sh.at[pl.ds(s * RPT, RPT)],
                    out_hbm.at[c].at[pl.ds(s * RPT, RPT)])


@functools.cache
def _sc_aggr():
    return pl.kernel(
        _sc_aggr_body,
        out_type=jax.ShapeDtypeStruct((2, NP, HH), jnp.float32),
        mesh=plsc.VectorSubcoreMesh(core_axis_name="c", subcore_axis_name="s",
                                    num_cores=NC, num_subcores=NS),
        compiler_params=pltpu.CompilerParams(use_tc_tiling_on_sc=False),
        scratch_types=[
            pltpu.VMEM((NCHUNK, CH), jnp.int32),
            pltpu.VMEM((NCHUNK, CH), jnp.int32),
            pltpu.VMEM((NBUF, CH, HH), jnp.float32),
            pltpu.VMEM_SHARED((NP, HH), jnp.float32),
            pltpu.SemaphoreType.DMA,
            pltpu.SemaphoreType.DMA,
        ],
    )


BN = 2000
NB = N // BN


def _mlp_body(x0_ref, x1_ref, p0_ref, p1_ref, W1_ref, b1_ref, W2_ref, b2_ref,
              z_ref, stats_ref):
    i = pl.program_id(0)
    x = jnp.concatenate([x0_ref[0], x1_ref[0]], axis=1)
    aggr = jnp.concatenate([p0_ref[0], p1_ref[0]], axis=1)
    h0 = x + aggr
    a = jnp.dot(h0, W1_ref[...], preferred_element_type=jnp.float32) + b1_ref[...]
    a = jnp.maximum(a, 0.0)
    z = jnp.dot(a, W2_ref[...], preferred_element_type=jnp.float32) + b2_ref[...]
    z = jnp.maximum(z, 0.0)
    z_ref[...] = z
    s = jnp.sum(z, axis=0, keepdims=True)
    q = jnp.sum(z * z, axis=0, keepdims=True)
    st = jnp.concatenate([s, q], axis=0)

    @pl.when(i == 0)
    def _():
        stats_ref[...] = st

    @pl.when(i > 0)
    def _():
        stats_ref[...] = stats_ref[...] + st


def _mlp_call(xs3, p, W1, b1, W2, b2):
    blk = pl.BlockSpec((BN, H), lambda i: (i, 0))
    half0 = pl.BlockSpec((1, BN, HH), lambda i: (0, i, 0))
    half1 = pl.BlockSpec((1, BN, HH), lambda i: (1, i, 0))
    full = lambda shape: pl.BlockSpec(shape, lambda i: (0,) * len(shape))
    return pl.pallas_call(
        _mlp_body,
        grid=(NB,),
        in_specs=[
            half0, half1, half0, half1,
            full((H, H)), full((1, H)), full((H, H)), full((1, H)),
        ],
        out_specs=[blk, full((2, H))],
        out_shape=[
            jax.ShapeDtypeStruct((N, H), jnp.float32),
            jax.ShapeDtypeStruct((2, H), jnp.float32),
        ],
    )(xs3, xs3, p, p, W1, b1.reshape(1, H), W2, b2.reshape(1, H))


def _norm_body(z_ref, st_ref, g_ref, b_ref, o_ref):
    mu = st_ref[0:1, :] / N
    var = st_ref[1:2, :] / N - mu * mu
    inv = lax.rsqrt(var + 1e-5)
    h = g_ref[...] * (z_ref[...] - mu) * inv + b_ref[...]
    o_ref[...] = jnp.stack([h[:, :HH], h[:, HH:]], axis=0)


def _norm_call(z, st, gamma, beta):
    blk = pl.BlockSpec((BN, H), lambda i: (i, 0))
    full = lambda shape: pl.BlockSpec(shape, lambda i: (0,) * len(shape))
    return pl.pallas_call(
        _norm_body,
        grid=(NB,),
        in_specs=[blk, full((2, H)), full((1, H)), full((1, H))],
        out_specs=pl.BlockSpec((2, BN, HH), lambda i: (0, i, 0)),
        out_shape=jax.ShapeDtypeStruct((2, N, HH), jnp.float32),
    )(z, st, gamma.reshape(1, H), beta.reshape(1, H))


def _final_body(z_ref, st_ref, g_ref, b_ref, W1_ref, b1_ref, W2_ref, b2_ref,
                o_ref):
    mu = st_ref[0:1, :] / N
    var = st_ref[1:2, :] / N - mu * mu
    h = g_ref[...] * (z_ref[...] - mu) * lax.rsqrt(var + 1e-5) + b_ref[...]
    h = jnp.dot(h, W1_ref[...], preferred_element_type=jnp.float32) + b1_ref[...]
    h = jnp.maximum(h, 0.0)
    o = jnp.dot(h, W2_ref[...], preferred_element_type=jnp.float32) + b2_ref[...]
    m = jnp.max(o, axis=1, keepdims=True)
    lse = jnp.log(jnp.sum(jnp.exp(o - m), axis=1, keepdims=True)) + m
    o_ref[...] = o - lse


def _final_call(z, st, gamma, beta, fc1_W, fc1_b, fc2_W, fc2_b):
    blk = pl.BlockSpec((BN, H), lambda i: (i, 0))
    full = lambda shape: pl.BlockSpec(shape, lambda i: (0,) * len(shape))
    return pl.pallas_call(
        _final_body,
        grid=(NB,),
        in_specs=[blk, full((2, H)), full((1, H)), full((1, H)),
                  full((H, H)), full((1, H)), full((H, C)), full((1, C))],
        out_specs=pl.BlockSpec((BN, C), lambda i: (i, 0)),
        out_shape=jax.ShapeDtypeStruct((N, C), jnp.float32),
    )(z, st, gamma.reshape(1, H), beta.reshape(1, H),
      fc1_W, fc1_b.reshape(1, H), fc2_W, fc2_b.reshape(1, C))


def kernel(x, edge_index, g0_W1, g0_b1, g0_W2, g0_b2, g1_W1, g1_b1, g1_W2,
           g1_b2, bn0_gamma, bn0_beta, bn1_gamma, bn1_beta, fc1_W, fc1_b,
           fc2_W, fc2_b):
    ei = edge_index.astype(jnp.int32)
    src_r = jnp.concatenate(
        [ei[0].reshape(NS, EPW), jnp.zeros((NS, PAD), jnp.int32)],
        axis=1).reshape(NS, NCHUNK, CH)
    src3 = jnp.concatenate([src_r, src_r + N], axis=0)
    dst3 = jnp.concatenate(
        [ei[1].reshape(NS, EPW), jnp.full((NS, PAD), N, jnp.int32)],
        axis=1).reshape(NS, NCHUNK, CH)
    zeros = jnp.zeros((RPT, HH), jnp.float32)

    xs3 = jnp.stack([x[:, :HH], x[:, HH:]], axis=0)

    p = _sc_aggr()(xs3.reshape(2 * N, HH), src3, dst3, zeros)
    z, st = _mlp_call(xs3, p, g0_W1, g0_b1, g0_W2, g0_b2)
    hs3 = _norm_call(z, st, bn0_gamma, bn0_beta)

    p = _sc_aggr()(hs3.reshape(2 * N, HH), src3, dst3, zeros)
    z1, st1 = _mlp_call(hs3, p, g1_W1, g1_b1, g1_W2, g1_b2)
    return _final_call(z1, st1, bn1_gamma, bn1_beta, fc1_W, fc1_b, fc2_W, fc2_b)


# speedup vs baseline: 1.3199x; 1.3199x over previous
import functools

import jax
import jax.numpy as jnp
from jax import lax
from jax.experimental import pallas as pl
from jax.experimental.pallas import tpu as pltpu
from jax.experimental.pallas import tpu_sc as plsc

N = 10000
E = 320000
H = 128
C = 10

NC = 2
NS = 16
NW = NC * NS
HH = H // NC
EPW = E // NS
CH = 80
NCHUNK = 250
NBUF = 5
PAD = NCHUNK * CH - EPW
NP = 10240
RPT = NP // NS


def _sc_aggr_body(x_hbm, src_hbm, dst_hbm, zeros_hbm, out_hbm,
                  src_v, dst_v, rows_v, aggr_sh, gsem, ssem):
    c = lax.axis_index("c")
    s = lax.axis_index("s")
    wid = c * NS + s

    pltpu.async_copy(src_hbm.at[wid], src_v, gsem)
    pltpu.async_copy(dst_hbm.at[s], dst_v, gsem)
    pltpu.sync_copy(zeros_hbm, aggr_sh.at[pl.ds(s * RPT, RPT)])
    pltpu.make_async_copy(src_hbm.at[wid], src_v, gsem).wait()
    pltpu.make_async_copy(dst_hbm.at[s], dst_v, gsem).wait()

    plsc.subcore_barrier()

    def start_gather(j, b):
        pltpu.async_copy(x_hbm.at[src_v.at[j]], rows_v.at[b], gsem)

    def wait_gather(b):
        pltpu.make_async_copy(x_hbm.at[pl.ds(0, CH)], rows_v.at[b], gsem).wait()

    def start_scatter(j, b):
        pltpu.async_copy(rows_v.at[b], aggr_sh.at[dst_v.at[j]], ssem, add=True)

    def wait_scatter(b):
        pltpu.make_async_copy(x_hbm.at[pl.ds(0, CH)], rows_v.at[b], ssem).wait()

    for b0 in range(NBUF - 1):
        start_gather(b0, b0)

    def outer(i, carry):
        for b in range(NBUF):
            j = NBUF * i + b
            wait_gather(b)
            start_scatter(j, b)
            jj = j + NBUF - 1

            @pl.when(jnp.logical_and(j >= 1, jj < NCHUNK))
            def _():
                wait_scatter((b + NBUF - 1) % NBUF)

            @pl.when(jj < NCHUNK)
            def _():
                start_gather(jj, (b + NBUF - 1) % NBUF)
        return carry

    lax.fori_loop(0, NCHUNK // NBUF, outer, 0)
    for _ in range(NBUF):
        wait_scatter(0)

    plsc.subcore_barrier()

    pltpu.sync_copy(aggr_sh.at[pl.ds(s * RPT, RPT)],
                    out_hbm.at[c].at[pl.ds(s * RPT, RPT)])


@functools.cache
def _sc_aggr():
    return pl.kernel(
        _sc_aggr_body,
        out_type=jax.ShapeDtypeStruct((2, NP, HH), jnp.float32),
        mesh=plsc.VectorSubcoreMesh(core_axis_name="c", subcore_axis_name="s",
                                    num_cores=NC, num_subcores=NS),
        compiler_params=pltpu.CompilerParams(use_tc_tiling_on_sc=False),
        scratch_types=[
            pltpu.VMEM((NCHUNK, CH), jnp.int32),
            pltpu.VMEM((NCHUNK, CH), jnp.int32),
            pltpu.VMEM((NBUF, CH, HH), jnp.float32),
            pltpu.VMEM_SHARED((NP, HH), jnp.float32),
            pltpu.SemaphoreType.DMA,
            pltpu.SemaphoreType.DMA,
        ],
    )


BN = 2000
NB = N // BN


def _mlp_body(x0_ref, x1_ref, p0_ref, p1_ref, W1_ref, b1_ref, W2_ref, b2_ref,
              z_ref, stats_ref):
    i = pl.program_id(0)
    x = jnp.concatenate([x0_ref[0], x1_ref[0]], axis=1)
    aggr = jnp.concatenate([p0_ref[0], p1_ref[0]], axis=1)
    h0 = x + aggr
    a = jnp.dot(h0, W1_ref[...], preferred_element_type=jnp.float32) + b1_ref[...]
    a = jnp.maximum(a, 0.0)
    z = jnp.dot(a, W2_ref[...], preferred_element_type=jnp.float32) + b2_ref[...]
    z = jnp.maximum(z, 0.0)
    z_ref[...] = z
    s = jnp.sum(z, axis=0, keepdims=True)
    q = jnp.sum(z * z, axis=0, keepdims=True)
    st = jnp.concatenate([s, q], axis=0)

    @pl.when(i == 0)
    def _():
        stats_ref[...] = st

    @pl.when(i > 0)
    def _():
        stats_ref[...] = stats_ref[...] + st


def _mlp_call(xs3, p, W1, b1, W2, b2):
    blk = pl.BlockSpec((BN, H), lambda i: (i, 0))
    half0 = pl.BlockSpec((1, BN, HH), lambda i: (0, i, 0))
    half1 = pl.BlockSpec((1, BN, HH), lambda i: (1, i, 0))
    full = lambda shape: pl.BlockSpec(shape, lambda i: (0,) * len(shape))
    return pl.pallas_call(
        _mlp_body,
        grid=(NB,),
        in_specs=[
            half0, half1, half0, half1,
            full((H, H)), full((1, H)), full((H, H)), full((1, H)),
        ],
        out_specs=[blk, full((2, H))],
        out_shape=[
            jax.ShapeDtypeStruct((N, H), jnp.float32),
            jax.ShapeDtypeStruct((2, H), jnp.float32),
        ],
    )(xs3, xs3, p, p, W1, b1.reshape(1, H), W2, b2.reshape(1, H))


def _norm_body(z_ref, st_ref, g_ref, b_ref, o_ref):
    mu = st_ref[0:1, :] / N
    var = st_ref[1:2, :] / N - mu * mu
    inv = lax.rsqrt(var + 1e-5)
    h = g_ref[...] * (z_ref[...] - mu) * inv + b_ref[...]
    o_ref[...] = jnp.stack([h[:, :HH], h[:, HH:]], axis=0)


def _norm_call(z, st, gamma, beta):
    blk = pl.BlockSpec((BN, H), lambda i: (i, 0))
    full = lambda shape: pl.BlockSpec(shape, lambda i: (0,) * len(shape))
    return pl.pallas_call(
        _norm_body,
        grid=(NB,),
        in_specs=[blk, full((2, H)), full((1, H)), full((1, H))],
        out_specs=pl.BlockSpec((2, BN, HH), lambda i: (0, i, 0)),
        out_shape=jax.ShapeDtypeStruct((2, N, HH), jnp.float32),
    )(z, st, gamma.reshape(1, H), beta.reshape(1, H))


def _final_body(z_ref, st_ref, g_ref, b_ref, W1_ref, b1_ref, W2_ref, b2_ref,
                o_ref):
    mu = st_ref[0:1, :] / N
    var = st_ref[1:2, :] / N - mu * mu
    h = g_ref[...] * (z_ref[...] - mu) * lax.rsqrt(var + 1e-5) + b_ref[...]
    h = jnp.dot(h, W1_ref[...], preferred_element_type=jnp.float32) + b1_ref[...]
    h = jnp.maximum(h, 0.0)
    o = jnp.dot(h, W2_ref[...], preferred_element_type=jnp.float32) + b2_ref[...]
    m = jnp.max(o, axis=1, keepdims=True)
    lse = jnp.log(jnp.sum(jnp.exp(o - m), axis=1, keepdims=True)) + m
    o_ref[...] = o - lse


def _final_call(z, st, gamma, beta, fc1_W, fc1_b, fc2_W, fc2_b):
    blk = pl.BlockSpec((BN, H), lambda i: (i, 0))
    full = lambda shape: pl.BlockSpec(shape, lambda i: (0,) * len(shape))
    return pl.pallas_call(
        _final_body,
        grid=(NB,),
        in_specs=[blk, full((2, H)), full((1, H)), full((1, H)),
                  full((H, H)), full((1, H)), full((H, C)), full((1, C))],
        out_specs=pl.BlockSpec((BN, C), lambda i: (i, 0)),
        out_shape=jax.ShapeDtypeStruct((N, C), jnp.float32),
    )(z, st, gamma.reshape(1, H), beta.reshape(1, H),
      fc1_W, fc1_b.reshape(1, H), fc2_W, fc2_b.reshape(1, C))


def kernel(x, edge_index, g0_W1, g0_b1, g0_W2, g0_b2, g1_W1, g1_b1, g1_W2,
           g1_b2, bn0_gamma, bn0_beta, bn1_gamma, bn1_beta, fc1_W, fc1_b,
           fc2_W, fc2_b):
    ei = edge_index.astype(jnp.int32)
    src_r = jnp.concatenate(
        [ei[0].reshape(NS, EPW), jnp.zeros((NS, PAD), jnp.int32)],
        axis=1).reshape(NS, NCHUNK, CH)
    src3 = jnp.concatenate([src_r, src_r + N], axis=0)
    dst3 = jnp.concatenate(
        [ei[1].reshape(NS, EPW), jnp.full((NS, PAD), N, jnp.int32)],
        axis=1).reshape(NS, NCHUNK, CH)
    zeros = jnp.zeros((RPT, HH), jnp.float32)

    xs3 = jnp.stack([x[:, :HH], x[:, HH:]], axis=0)

    p = _sc_aggr()(xs3.reshape(2 * N, HH), src3, dst3, zeros)
    z, st = _mlp_call(xs3, p, g0_W1, g0_b1, g0_W2, g0_b2)
    hs3 = _norm_call(z, st, bn0_gamma, bn0_beta)

    p = _sc_aggr()(hs3.reshape(2 * N, HH), src3, dst3, zeros)
    z1, st1 = _mlp_call(hs3, p, g1_W1, g1_b1, g1_W2, g1_b2)
    return _final_call(z1, st1, bn1_gamma, bn1_beta, fc1_W, fc1_b, fc2_W, fc2_b)
